# baseline (speedup 1.0000x reference)
import functools

import jax
import jax.numpy as jnp
from jax import lax
from jax.experimental import pallas as pl
from jax.experimental.pallas import tpu as pltpu
from jax.experimental.pallas import tpu_sc as plsc

H = 128
NG = 16
ZC = 2 * H

SC_NC = 2
SC_NS = 16
NWORK = SC_NC * SC_NS

CE = 40
NBUF = 4

NP_PAD = 10240
NL_PAD = 2048
E_PP_PAD = 163840
E_LP_PAD = 61440
E_PL_PAD = 61440


def _pad_rows(x, n):
    return jnp.pad(x, ((0, n - x.shape[0]), (0, 0)))


def _sc_gather_body(chunks, pd_pp, ps_pp, pd_lp, ps_lp, pd_pl, ps_pl,
                    d_pp, s_pp, d_lp, s_lp, d_pl, s_pl,
                    z_pp, z_lp, z_pl,
                    idxd, idxs, a_v, b_v, o_v,
                    sema, semb, semo):
    cid = lax.axis_index("c")
    sid = lax.axis_index("s")
    wid = sid * SC_NC + cid

    for (nch, dj, sj, pd, ps, zout) in (
            (chunks[0], d_pp, s_pp, pd_pp, ps_pp, z_pp),
            (chunks[1], d_lp, s_lp, pd_lp, ps_lp, z_lp),
            (chunks[2], d_pl, s_pl, pd_pl, ps_pl, z_pl)):
        ngroups = nch // NBUF
        pltpu.sync_copy(dj.at[wid], idxd.at[pl.ds(0, nch)])
        pltpu.sync_copy(sj.at[wid], idxs.at[pl.ds(0, nch)])

        for sl in range(NBUF):
            pltpu.async_copy(pd.at[idxd.at[sl]], a_v[sl], sema.at[sl])
            pltpu.async_copy(ps.at[idxs.at[sl]], b_v[sl], semb.at[sl])

        def group_body(g, _):
            for sl in range(NBUF):
                c = g * NBUF + sl
                pltpu.make_async_copy(pd.at[idxd.at[sl]], a_v[sl],
                                      sema.at[sl]).wait()
                pltpu.make_async_copy(ps.at[idxs.at[sl]], b_v[sl],
                                      semb.at[sl]).wait()

                @pl.when(g > 0)
                def _():
                    pltpu.make_async_copy(o_v[sl], zout.at[pl.ds(0, CE)],
                                          semo.at[sl]).wait()

                def copy_body(e, _):
                    for v in range(8):
                        vsl = pl.ds(v * 16, 16)
                        o_v[sl][e, vsl] = a_v[sl][e, vsl]
                        o_v[sl][e, pl.ds(H + v * 16, 16)] = b_v[sl][e, vsl]
                    return 0

                lax.fori_loop(0, CE, copy_body, 0)

                @pl.when(c + NBUF < nch)
                def _():
                    pltpu.async_copy(pd.at[idxd.at[c + NBUF]], a_v[sl],
                                     sema.at[sl])
                    pltpu.async_copy(ps.at[idxs.at[c + NBUF]], b_v[sl],
                                     semb.at[sl])

                row0 = (wid * nch + c) * CE
                pltpu.async_copy(o_v[sl], zout.at[pl.ds(row0, CE)],
                                 semo.at[sl])
            return 0

        lax.fori_loop(0, ngroups, group_body, 0)

        for sl in range(NBUF):
            pltpu.make_async_copy(o_v[sl], zout.at[pl.ds(0, CE)],
                                  semo.at[sl]).wait()


def _sc_gather(tables, idxs2d, e_pads):
    chunks = tuple(e // (NWORK * CE) for e in e_pads)
    nch_max = max(chunks)
    out_type = [jax.ShapeDtypeStruct((e, 2 * H), jnp.int32) for e in e_pads]
    f = pl.kernel(
        functools.partial(_sc_gather_body, chunks),
        out_type=out_type,
        mesh=plsc.VectorSubcoreMesh(core_axis_name="c", subcore_axis_name="s",
                                    num_cores=SC_NC, num_subcores=SC_NS),
        scratch_types=[
            pltpu.VMEM((nch_max, CE), jnp.int32),
            pltpu.VMEM((nch_max, CE), jnp.int32),
            [pltpu.VMEM((CE, H), jnp.int32)] * NBUF,
            [pltpu.VMEM((CE, H), jnp.int32)] * NBUF,
            [pltpu.VMEM((CE, 2 * H), jnp.int32)] * NBUF,
            pltpu.SemaphoreType.DMA((NBUF,)),
            pltpu.SemaphoreType.DMA((NBUF,)),
            pltpu.SemaphoreType.DMA((NBUF,)),
        ],
    )
    return f(*tables, *idxs2d)


CS = 80
AGG_ROWS = NP_PAD + NL_PAD


def _sc_scatter_body(msg_pp, msg_lp, msg_pl, d_pp, d_lp, d_pl,
                     agg_pp, agg_lp, agg_pl,
                     spmem, zbuf, mbuf, ibuf, sem):
    cid = lax.axis_index("c")
    tid = lax.axis_index("s")

    def zrow(e, _):
        for v in range(H // 16):
            zbuf[e, pl.ds(v * 16, 16)] = jnp.zeros((16,), jnp.float32)
        return 0
    lax.fori_loop(0, 64, zrow, 0)
    nz = AGG_ROWS // (16 * 64)
    for k in range(nz):
        pltpu.sync_copy(zbuf, spmem.at[pl.ds((tid * nz + k) * 64, 64)])
    plsc.subcore_barrier()

    def scatter_edges(msg, dj, nch):
        def chunk_body(c, _):
            base = (tid * nch + c) * CS
            cpi = pltpu.async_copy(dj.at[pl.ds(base, CS)], ibuf, sem)
            cpm = pltpu.async_copy(msg.at[pl.ds(base, CS)], mbuf, sem)
            cpi.wait()
            cpm.wait()
            pltpu.sync_copy(mbuf, spmem.at[ibuf], add=True)
            return 0
        lax.fori_loop(0, nch, chunk_body, 0)

    @pl.when(cid == 0)
    def _():
        scatter_edges(msg_pp, d_pp, E_PP_PAD // (SC_NS * CS))

    @pl.when(cid == 1)
    def _():
        scatter_edges(msg_lp, d_lp, E_LP_PAD // (SC_NS * CS))
        scatter_edges(msg_pl, d_pl, E_PL_PAD // (SC_NS * CS))

    plsc.subcore_barrier()

    def writeout(out_hbm, spm_base, rows_per_tile):
        for k in range(rows_per_tile // 64):
            r0 = spm_base + tid * rows_per_tile + k * 64
            o0 = tid * rows_per_tile + k * 64
            pltpu.sync_copy(spmem.at[pl.ds(r0, 64)], zbuf)
            pltpu.sync_copy(zbuf, out_hbm.at[pl.ds(o0, 64)])

    @pl.when(cid == 0)
    def _():
        writeout(agg_pp, 0, NP_PAD // SC_NS)

    @pl.when(cid == 1)
    def _():
        writeout(agg_lp, 0, NP_PAD // SC_NS)
        writeout(agg_pl, NP_PAD, NL_PAD // SC_NS)


def _sc_scatter(msg_pp, msg_lp, msg_pl, d_pp, d_lp, d_pl):
    out_type = [jax.ShapeDtypeStruct((NP_PAD, H), jnp.float32),
                jax.ShapeDtypeStruct((NP_PAD, H), jnp.float32),
                jax.ShapeDtypeStruct((NL_PAD, H), jnp.float32)]
    f = pl.kernel(
        _sc_scatter_body,
        out_type=out_type,
        mesh=plsc.VectorSubcoreMesh(core_axis_name="c", subcore_axis_name="s",
                                    num_cores=SC_NC, num_subcores=SC_NS),
        scratch_types=[
            pltpu.VMEM_SHARED((AGG_ROWS, H), jnp.float32),
            pltpu.VMEM((64, H), jnp.float32),
            pltpu.VMEM((CS, H), jnp.float32),
            pltpu.VMEM((CS,), jnp.int32),
            pltpu.SemaphoreType.DMA,
        ],
    )
    return f(msg_pp, msg_lp, msg_pl, d_pp, d_lp, d_pl)


def _gate_body(zab_ref, ea_ref, we_ref, o_ref):
    def lo(w):
        return jax.lax.bitcast_convert_type(w << 16, jnp.float32)

    def hi(w):
        return jax.lax.bitcast_convert_type(w & jnp.int32(-65536), jnp.float32)

    za = zab_ref[:, :H]
    zb = zab_ref[:, H:]
    ec = jnp.dot(ea_ref[...], we_ref[...], preferred_element_type=jnp.float32)
    zf = lo(za) + lo(zb) + ec[:, :H]
    zs = hi(za) + hi(zb) + ec[:, H:]
    o_ref[...] = jax.nn.sigmoid(zf) * jax.nn.softplus(zs)


def _gate(zab, ea, wecat, blk):
    e = zab.shape[0]
    grid = e // blk
    return pl.pallas_call(
        _gate_body,
        grid=(grid,),
        in_specs=[pl.BlockSpec((blk, 2 * H), lambda i: (i, 0)),
                  pl.BlockSpec((blk, NG), lambda i: (i, 0)),
                  pl.BlockSpec((NG, ZC), lambda i: (0, 0))],
        out_specs=pl.BlockSpec((blk, H), lambda i: (i, 0)),
        out_shape=jax.ShapeDtypeStruct((e, H), jnp.float32),
    )(zab, ea, wecat)


def _smear(dist):
    offset = jnp.linspace(0.0, 8.0, NG)
    coeff = -0.5 / ((8.0 - 0.0) / (NG - 1)) ** 2
    d = dist.reshape(-1, 1) - offset.reshape(1, -1)
    return jnp.exp(coeff * d * d)


def _ln(x, g, b):
    mu = jnp.mean(x, axis=-1, keepdims=True)
    var = jnp.var(x, axis=-1, keepdims=True)
    return (x - mu) / jnp.sqrt(var + 1e-5) * g + b


def _post(agg, x_dst, p):
    agg = agg / jnp.sqrt(1.0 + 1e-5) * p['bn_g'] + p['bn_b']
    out = jax.nn.relu(_ln(agg + x_dst, p['ln_g'], p['ln_b']))
    return out + x_dst


def _proj(x, p, part):
    lo = 0 if part == 'i' else H
    w = jnp.concatenate([p['Wf'][lo:lo + H], p['Ws'][lo:lo + H]], axis=1)
    t = x @ w
    if part == 'i':
        t = t + jnp.concatenate([p['bf'], p['bs']])
    fu = jax.lax.bitcast_convert_type(t[:, :H].astype(jnp.bfloat16),
                                      jnp.uint16).astype(jnp.uint32)
    su = jax.lax.bitcast_convert_type(t[:, H:].astype(jnp.bfloat16),
                                      jnp.uint16).astype(jnp.uint32)
    return jax.lax.bitcast_convert_type((su << 16) | fu, jnp.int32)


def _pad_edges(ei, n_pad, dst_pad_base):
    e = ei.shape[1]
    if e == n_pad:
        return ei[0], ei[1]
    pad = n_pad - e
    src = jnp.concatenate([ei[0], jnp.zeros((pad,), jnp.int32)])
    dst = jnp.concatenate(
        [ei[1], dst_pad_base + (jnp.arange(pad, dtype=jnp.int32) % 8)])
    return src, dst


def kernel(x_protein, x_ligand, edge_index_pp, edge_index_lp, edge_index_pl,
           edge_attr_pp, edge_attr_lp, edge_attr_pl, params):
    hp = _pad_rows(x_protein, NP_PAD) @ params['Wp'] + params['bp']
    hl = _pad_rows(x_ligand, NL_PAD) @ params['Wl'] + params['bl']

    src_pp, dst_pp = _pad_edges(edge_index_pp, E_PP_PAD, 10000)
    src_lp, dst_lp = _pad_edges(edge_index_lp, E_LP_PAD, 10000)
    src_pl, dst_pl = _pad_edges(edge_index_pl, E_PL_PAD, 2000)

    ea_pp = _smear(jnp.pad(edge_attr_pp, (0, E_PP_PAD - edge_attr_pp.shape[0])))
    ea_lp = _smear(jnp.pad(edge_attr_lp, (0, E_LP_PAD - edge_attr_lp.shape[0])))
    ea_pl = _smear(jnp.pad(edge_attr_pl, (0, E_PL_PAD - edge_attr_pl.shape[0])))

    dst_pl_off = dst_pl + NP_PAD

    idxs2d = [x.reshape(NWORK, -1, CE) for x in
              (dst_pp, src_pp, dst_lp, src_lp, dst_pl, src_pl)]
    e_pads = (E_PP_PAD, E_LP_PAD, E_PL_PAD)

    for lp in params['layers']:
        tables = (_proj(hp, lp['pp'], 'i'), _proj(hp, lp['pp'], 'j'),
                  _proj(hp, lp['lp'], 'i'), _proj(hl, lp['lp'], 'j'),
                  _proj(hl, lp['pl'], 'i'), _proj(hp, lp['pl'], 'j'))
        z_pp, z_lp, z_pl = _sc_gather(tables, idxs2d, e_pads)

        we_pp = jnp.concatenate([lp['pp']['Wf'][2 * H:], lp['pp']['Ws'][2 * H:]], 1)
        we_lp = jnp.concatenate([lp['lp']['Wf'][2 * H:], lp['lp']['Ws'][2 * H:]], 1)
        we_pl = jnp.concatenate([lp['pl']['Wf'][2 * H:], lp['pl']['Ws'][2 * H:]], 1)

        msg_pp = _gate(z_pp, ea_pp, we_pp, 640)
        msg_lp = _gate(z_lp, ea_lp, we_lp, 640)
        msg_pl = _gate(z_pl, ea_pl, we_pl, 640)

        agg_pp, agg_lp, agg_pl = _sc_scatter(
            msg_pp, msg_lp, msg_pl, dst_pp, dst_lp, dst_pl_off)

        new_p = _post(agg_pp, hp, lp['pp']) + _post(agg_lp, hp, lp['lp'])
        new_l = _post(agg_pl, hl, lp['pl'])
        hp, hl = new_p, new_l

    hp = _ln(hp[:10000], params['lno_g'], params['lno_b'])
    return hp @ params['fcW'] + params['fcb']


# speedup vs baseline: 1.5056x; 1.5056x over previous
import functools

import jax
import jax.numpy as jnp
from jax import lax
from jax.experimental import pallas as pl
from jax.experimental.pallas import tpu as pltpu
from jax.experimental.pallas import tpu_sc as plsc

H = 128
NG = 16
ZC = 2 * H

SC_NC = 2
SC_NS = 16
NWORK = SC_NC * SC_NS

CE = 40
NBUF = 3

NP_PAD = 10240
NL_PAD = 2048
E_PP_PAD = 161280
E_LP_PAD = 53760
E_PL_PAD = 53760


def _pad_rows(x, n):
    return jnp.pad(x, ((0, n - x.shape[0]), (0, 0)))


def _sc_gather_body(chunks, pd_pp, ps_pp, pd_lp, ps_lp, pd_pl, ps_pl,
                    d_pp, s_pp, d_lp, s_lp, d_pl, s_pl,
                    z_pp, z_lp, z_pl,
                    idxd, idxs, a_v, b_v, o_v, sema, semb, semo):
    cid = lax.axis_index("c")
    sid = lax.axis_index("s")
    wid = sid * SC_NC + cid

    for (nch, dj, sj, pd, ps, zout) in (
            (chunks[0], d_pp, s_pp, pd_pp, ps_pp, z_pp),
            (chunks[1], d_lp, s_lp, pd_lp, ps_lp, z_lp),
            (chunks[2], d_pl, s_pl, pd_pl, ps_pl, z_pl)):
        ngroups = nch // NBUF
        pltpu.sync_copy(dj.at[wid], idxd.at[pl.ds(0, nch)])
        pltpu.sync_copy(sj.at[wid], idxs.at[pl.ds(0, nch)])

        for sl in range(NBUF):
            pltpu.async_copy(pd.at[idxd.at[sl]], a_v[sl], sema.at[sl])
            pltpu.async_copy(ps.at[idxs.at[sl]], b_v[sl], semb.at[sl])

        def group_body(g, _):
            for sl in range(NBUF):
                c = g * NBUF + sl
                pltpu.make_async_copy(pd.at[idxd.at[sl]], a_v[sl],
                                      sema.at[sl]).wait()
                pltpu.make_async_copy(ps.at[idxs.at[sl]], b_v[sl],
                                      semb.at[sl]).wait()

                @pl.when(g > 0)
                def _():
                    pltpu.make_async_copy(o_v[sl], zout.at[pl.ds(0, CE)],
                                          semo.at[sl]).wait()

                def add_body(e, _):
                    for v in range(ZC // 16):
                        vsl = pl.ds(v * 16, 16)
                        o_v[sl][e, vsl] = a_v[sl][e, vsl] + b_v[sl][e, vsl]
                    return 0

                lax.fori_loop(0, CE, add_body, 0)

                @pl.when(c + NBUF < nch)
                def _():
                    pltpu.async_copy(pd.at[idxd.at[c + NBUF]], a_v[sl],
                                     sema.at[sl])
                    pltpu.async_copy(ps.at[idxs.at[c + NBUF]], b_v[sl],
                                     semb.at[sl])

                row0 = (wid * nch + c) * CE
                pltpu.async_copy(o_v[sl], zout.at[pl.ds(row0, CE)],
                                 semo.at[sl])
            return 0

        lax.fori_loop(0, ngroups, group_body, 0)

        for sl in range(NBUF):
            pltpu.make_async_copy(o_v[sl], zout.at[pl.ds(0, CE)],
                                  semo.at[sl]).wait()


def _sc_gather(tables, idxs2d, e_pads):
    chunks = tuple(e // (NWORK * CE) for e in e_pads)
    nch_max = max(chunks)
    out_type = [jax.ShapeDtypeStruct((e, ZC), jnp.float32) for e in e_pads]
    f = pl.kernel(
        functools.partial(_sc_gather_body, chunks),
        out_type=out_type,
        mesh=plsc.VectorSubcoreMesh(core_axis_name="c", subcore_axis_name="s",
                                    num_cores=SC_NC, num_subcores=SC_NS),
        scratch_types=[
            pltpu.VMEM((nch_max, CE), jnp.int32),
            pltpu.VMEM((nch_max, CE), jnp.int32),
            [pltpu.VMEM((CE, ZC), jnp.float32)] * NBUF,
            [pltpu.VMEM((CE, ZC), jnp.float32)] * NBUF,
            [pltpu.VMEM((CE, ZC), jnp.float32)] * NBUF,
            pltpu.SemaphoreType.DMA((NBUF,)),
            pltpu.SemaphoreType.DMA((NBUF,)),
            pltpu.SemaphoreType.DMA((NBUF,)),
        ],
    )
    return f(*tables, *idxs2d)


CS = 80
AGG_ROWS = NP_PAD + NL_PAD


def _sc_scatter_body(msg_pp, msg_lp, msg_pl, d_pp, d_lp, d_pl,
                     agg_pp, agg_lp, agg_pl,
                     spmem, zbuf, mbuf, ibuf, sem):
    cid = lax.axis_index("c")
    tid = lax.axis_index("s")

    def zrow(e, _):
        for v in range(H // 16):
            zbuf[e, pl.ds(v * 16, 16)] = jnp.zeros((16,), jnp.float32)
        return 0
    lax.fori_loop(0, 64, zrow, 0)
    nz = AGG_ROWS // (16 * 64)
    for k in range(nz):
        pltpu.sync_copy(zbuf, spmem.at[pl.ds((tid * nz + k) * 64, 64)])
    plsc.subcore_barrier()

    def scatter_edges(msg, dj, nch):
        def chunk_body(c, _):
            base = (tid * nch + c) * CS
            cpi = pltpu.async_copy(dj.at[pl.ds(base, CS)], ibuf, sem)
            cpm = pltpu.async_copy(msg.at[pl.ds(base, CS)], mbuf, sem)
            cpi.wait()
            cpm.wait()
            pltpu.sync_copy(mbuf, spmem.at[ibuf], add=True)
            return 0
        lax.fori_loop(0, nch, chunk_body, 0)

    @pl.when(cid == 0)
    def _():
        scatter_edges(msg_pp, d_pp, E_PP_PAD // (SC_NS * CS))

    @pl.when(cid == 1)
    def _():
        scatter_edges(msg_lp, d_lp, E_LP_PAD // (SC_NS * CS))
        scatter_edges(msg_pl, d_pl, E_PL_PAD // (SC_NS * CS))

    plsc.subcore_barrier()

    def writeout(out_hbm, spm_base, rows_per_tile):
        for k in range(rows_per_tile // 64):
            r0 = spm_base + tid * rows_per_tile + k * 64
            o0 = tid * rows_per_tile + k * 64
            pltpu.sync_copy(spmem.at[pl.ds(r0, 64)], zbuf)
            pltpu.sync_copy(zbuf, out_hbm.at[pl.ds(o0, 64)])

    @pl.when(cid == 0)
    def _():
        writeout(agg_pp, 0, NP_PAD // SC_NS)

    @pl.when(cid == 1)
    def _():
        writeout(agg_lp, 0, NP_PAD // SC_NS)
        writeout(agg_pl, NP_PAD, NL_PAD // SC_NS)


def _sc_scatter(msg_pp, msg_lp, msg_pl, d_pp, d_lp, d_pl):
    out_type = [jax.ShapeDtypeStruct((NP_PAD, H), jnp.float32),
                jax.ShapeDtypeStruct((NP_PAD, H), jnp.float32),
                jax.ShapeDtypeStruct((NL_PAD, H), jnp.float32)]
    f = pl.kernel(
        _sc_scatter_body,
        out_type=out_type,
        mesh=plsc.VectorSubcoreMesh(core_axis_name="c", subcore_axis_name="s",
                                    num_cores=SC_NC, num_subcores=SC_NS),
        scratch_types=[
            pltpu.VMEM_SHARED((AGG_ROWS, H), jnp.float32),
            pltpu.VMEM((64, H), jnp.float32),
            pltpu.VMEM((CS, H), jnp.float32),
            pltpu.VMEM((CS,), jnp.int32),
            pltpu.SemaphoreType.DMA,
        ],
    )
    return f(msg_pp, msg_lp, msg_pl, d_pp, d_lp, d_pl)


def _gate_body(z_ref, ea_ref, we_ref, o_ref):
    z = z_ref[...] + jnp.dot(ea_ref[...], we_ref[...],
                             preferred_element_type=jnp.float32)
    zf = z[:, :H]
    zs = z[:, H:]
    o_ref[...] = jax.nn.sigmoid(zf) * jax.nn.softplus(zs)


def _gate(zcat, ea, wecat, blk):
    e = zcat.shape[0]
    grid = e // blk
    return pl.pallas_call(
        _gate_body,
        grid=(grid,),
        in_specs=[pl.BlockSpec((blk, ZC), lambda i: (i, 0)),
                  pl.BlockSpec((blk, NG), lambda i: (i, 0)),
                  pl.BlockSpec((NG, ZC), lambda i: (0, 0))],
        out_specs=pl.BlockSpec((blk, H), lambda i: (i, 0)),
        out_shape=jax.ShapeDtypeStruct((e, H), jnp.float32),
    )(zcat, ea, wecat)


def _smear(dist):
    offset = jnp.linspace(0.0, 8.0, NG)
    coeff = -0.5 / ((8.0 - 0.0) / (NG - 1)) ** 2
    d = dist.reshape(-1, 1) - offset.reshape(1, -1)
    return jnp.exp(coeff * d * d)


def _ln(x, g, b):
    mu = jnp.mean(x, axis=-1, keepdims=True)
    var = jnp.var(x, axis=-1, keepdims=True)
    return (x - mu) / jnp.sqrt(var + 1e-5) * g + b


def _post(agg, x_dst, p):
    agg = agg / jnp.sqrt(1.0 + 1e-5) * p['bn_g'] + p['bn_b']
    out = jax.nn.relu(_ln(agg + x_dst, p['ln_g'], p['ln_b']))
    return out + x_dst


def _proj(x, p, part):
    lo = 0 if part == 'i' else H
    w = jnp.concatenate([p['Wf'][lo:lo + H], p['Ws'][lo:lo + H]], axis=1)
    t = x @ w
    if part == 'i':
        t = t + jnp.concatenate([p['bf'], p['bs']])
    return t


def _pad_edges(ei, n_pad, dst_pad_base):
    e = ei.shape[1]
    if e == n_pad:
        return ei[0], ei[1]
    pad = n_pad - e
    src = jnp.concatenate([ei[0], jnp.zeros((pad,), jnp.int32)])
    dst = jnp.concatenate(
        [ei[1], dst_pad_base + (jnp.arange(pad, dtype=jnp.int32) % 8)])
    return src, dst


def kernel(x_protein, x_ligand, edge_index_pp, edge_index_lp, edge_index_pl,
           edge_attr_pp, edge_attr_lp, edge_attr_pl, params):
    hp = _pad_rows(x_protein, NP_PAD) @ params['Wp'] + params['bp']
    hl = _pad_rows(x_ligand, NL_PAD) @ params['Wl'] + params['bl']

    src_pp, dst_pp = _pad_edges(edge_index_pp, E_PP_PAD, 10000)
    src_lp, dst_lp = _pad_edges(edge_index_lp, E_LP_PAD, 10000)
    src_pl, dst_pl = _pad_edges(edge_index_pl, E_PL_PAD, 2000)

    ea_pp = _smear(jnp.pad(edge_attr_pp, (0, E_PP_PAD - edge_attr_pp.shape[0])))
    ea_lp = _smear(jnp.pad(edge_attr_lp, (0, E_LP_PAD - edge_attr_lp.shape[0])))
    ea_pl = _smear(jnp.pad(edge_attr_pl, (0, E_PL_PAD - edge_attr_pl.shape[0])))

    dst_pl_off = dst_pl + NP_PAD

    idxs2d = [x.reshape(NWORK, -1, CE) for x in
              (dst_pp, src_pp, dst_lp, src_lp, dst_pl, src_pl)]
    e_pads = (E_PP_PAD, E_LP_PAD, E_PL_PAD)

    for lp in params['layers']:
        tables = (_proj(hp, lp['pp'], 'i'), _proj(hp, lp['pp'], 'j'),
                  _proj(hp, lp['lp'], 'i'), _proj(hl, lp['lp'], 'j'),
                  _proj(hl, lp['pl'], 'i'), _proj(hp, lp['pl'], 'j'))
        z_pp, z_lp, z_pl = _sc_gather(tables, idxs2d, e_pads)

        we_pp = jnp.concatenate([lp['pp']['Wf'][2 * H:], lp['pp']['Ws'][2 * H:]], 1)
        we_lp = jnp.concatenate([lp['lp']['Wf'][2 * H:], lp['lp']['Ws'][2 * H:]], 1)
        we_pl = jnp.concatenate([lp['pl']['Wf'][2 * H:], lp['pl']['Ws'][2 * H:]], 1)

        msg_pp = _gate(z_pp, ea_pp, we_pp, 640)
        msg_lp = _gate(z_lp, ea_lp, we_lp, 640)
        msg_pl = _gate(z_pl, ea_pl, we_pl, 640)

        agg_pp, agg_lp, agg_pl = _sc_scatter(
            msg_pp, msg_lp, msg_pl, dst_pp, dst_lp, dst_pl_off)

        new_p = _post(agg_pp, hp, lp['pp']) + _post(agg_lp, hp, lp['lp'])
        new_l = _post(agg_pl, hl, lp['pl'])
        hp, hl = new_p, new_l

    hp = _ln(hp[:10000], params['lno_g'], params['lno_b'])
    return hp @ params['fcW'] + params['fcb']


# speedup vs baseline: 1.5815x; 1.0504x over previous
import functools

import jax
import jax.numpy as jnp
from jax import lax
from jax.experimental import pallas as pl
from jax.experimental.pallas import tpu as pltpu
from jax.experimental.pallas import tpu_sc as plsc

H = 128
NG = 16
ZC = 2 * H

SC_NC = 2
SC_NS = 16
NWORK = SC_NC * SC_NS

CE = 40
NBUF = 3

NP_PAD = 10240
NL_PAD = 2048
E_PP_PAD = 161280
E_LP_PAD = 53760
E_PL_PAD = 53760


def _pad_rows(x, n):
    return jnp.pad(x, ((0, n - x.shape[0]), (0, 0)))


def _sc_gather_body(nch, pd, ps, dj, sj, zout,
                    idxd, idxs, a_v, b_v, o_v, sema, semb, semo):
    cid = lax.axis_index("c")
    sid = lax.axis_index("s")
    wid = sid * SC_NC + cid

    ngroups = nch // NBUF
    pltpu.sync_copy(dj.at[wid], idxd)
    pltpu.sync_copy(sj.at[wid], idxs)

    for sl in range(NBUF):
        pltpu.async_copy(pd.at[idxd.at[sl]], a_v[sl], sema.at[sl])
        pltpu.async_copy(ps.at[idxs.at[sl]], b_v[sl], semb.at[sl])

    def group_body(g, _):
        for sl in range(NBUF):
            c = g * NBUF + sl
            pltpu.make_async_copy(pd.at[idxd.at[sl]], a_v[sl],
                                  sema.at[sl]).wait()
            pltpu.make_async_copy(ps.at[idxs.at[sl]], b_v[sl],
                                  semb.at[sl]).wait()

            @pl.when(g > 0)
            def _():
                pltpu.make_async_copy(o_v[sl], zout.at[pl.ds(0, CE)],
                                      semo.at[sl]).wait()

            def add_body(e, _):
                for v in range(ZC // 16):
                    vsl = pl.ds(v * 16, 16)
                    o_v[sl][e, vsl] = a_v[sl][e, vsl] + b_v[sl][e, vsl]
                return 0

            lax.fori_loop(0, CE, add_body, 0)

            @pl.when(c + NBUF < nch)
            def _():
                pltpu.async_copy(pd.at[idxd.at[c + NBUF]], a_v[sl],
                                 sema.at[sl])
                pltpu.async_copy(ps.at[idxs.at[c + NBUF]], b_v[sl],
                                 semb.at[sl])

            row0 = (wid * nch + c) * CE
            pltpu.async_copy(o_v[sl], zout.at[pl.ds(row0, CE)],
                             semo.at[sl])
        return 0

    lax.fori_loop(0, ngroups, group_body, 0)

    for sl in range(NBUF):
        pltpu.make_async_copy(o_v[sl], zout.at[pl.ds(0, CE)],
                              semo.at[sl]).wait()


def _sc_gather1(pd, ps, dj, sj, e_pad):
    nch = e_pad // (NWORK * CE)
    f = pl.kernel(
        functools.partial(_sc_gather_body, nch),
        out_type=jax.ShapeDtypeStruct((e_pad, ZC), jnp.float32),
        mesh=plsc.VectorSubcoreMesh(core_axis_name="c", subcore_axis_name="s",
                                    num_cores=SC_NC, num_subcores=SC_NS),
        scratch_types=[
            pltpu.VMEM((nch, CE), jnp.int32),
            pltpu.VMEM((nch, CE), jnp.int32),
            [pltpu.VMEM((CE, ZC), jnp.float32)] * NBUF,
            [pltpu.VMEM((CE, ZC), jnp.float32)] * NBUF,
            [pltpu.VMEM((CE, ZC), jnp.float32)] * NBUF,
            pltpu.SemaphoreType.DMA((NBUF,)),
            pltpu.SemaphoreType.DMA((NBUF,)),
            pltpu.SemaphoreType.DMA((NBUF,)),
        ],
    )
    return f(pd, ps, dj, sj)


CS = 80
AGG_ROWS = NP_PAD + NL_PAD


def _sc_scatter_body(msg_pp, msg_lp, msg_pl, d_pp, d_lp, d_pl,
                     agg_pp, agg_lp, agg_pl,
                     spmem, zbuf, mbuf, ibuf, sem):
    cid = lax.axis_index("c")
    tid = lax.axis_index("s")

    def zrow(e, _):
        for v in range(H // 16):
            zbuf[e, pl.ds(v * 16, 16)] = jnp.zeros((16,), jnp.float32)
        return 0
    lax.fori_loop(0, 64, zrow, 0)
    nz = AGG_ROWS // (16 * 64)
    for k in range(nz):
        pltpu.sync_copy(zbuf, spmem.at[pl.ds((tid * nz + k) * 64, 64)])
    plsc.subcore_barrier()

    def scatter_edges(msg, dj, nch):
        def chunk_body(c, _):
            base = (tid * nch + c) * CS
            cpi = pltpu.async_copy(dj.at[pl.ds(base, CS)], ibuf, sem)
            cpm = pltpu.async_copy(msg.at[pl.ds(base, CS)], mbuf, sem)
            cpi.wait()
            cpm.wait()
            pltpu.sync_copy(mbuf, spmem.at[ibuf], add=True)
            return 0
        lax.fori_loop(0, nch, chunk_body, 0)

    @pl.when(cid == 0)
    def _():
        scatter_edges(msg_pp, d_pp, E_PP_PAD // (SC_NS * CS))

    @pl.when(cid == 1)
    def _():
        scatter_edges(msg_lp, d_lp, E_LP_PAD // (SC_NS * CS))
        scatter_edges(msg_pl, d_pl, E_PL_PAD // (SC_NS * CS))

    plsc.subcore_barrier()

    def writeout(out_hbm, spm_base, rows_per_tile):
        for k in range(rows_per_tile // 64):
            r0 = spm_base + tid * rows_per_tile + k * 64
            o0 = tid * rows_per_tile + k * 64
            pltpu.sync_copy(spmem.at[pl.ds(r0, 64)], zbuf)
            pltpu.sync_copy(zbuf, out_hbm.at[pl.ds(o0, 64)])

    @pl.when(cid == 0)
    def _():
        writeout(agg_pp, 0, NP_PAD // SC_NS)

    @pl.when(cid == 1)
    def _():
        writeout(agg_lp, 0, NP_PAD // SC_NS)
        writeout(agg_pl, NP_PAD, NL_PAD // SC_NS)


def _sc_scatter(msg_pp, msg_lp, msg_pl, d_pp, d_lp, d_pl):
    out_type = [jax.ShapeDtypeStruct((NP_PAD, H), jnp.float32),
                jax.ShapeDtypeStruct((NP_PAD, H), jnp.float32),
                jax.ShapeDtypeStruct((NL_PAD, H), jnp.float32)]
    f = pl.kernel(
        _sc_scatter_body,
        out_type=out_type,
        mesh=plsc.VectorSubcoreMesh(core_axis_name="c", subcore_axis_name="s",
                                    num_cores=SC_NC, num_subcores=SC_NS),
        scratch_types=[
            pltpu.VMEM_SHARED((AGG_ROWS, H), jnp.float32),
            pltpu.VMEM((64, H), jnp.float32),
            pltpu.VMEM((CS, H), jnp.float32),
            pltpu.VMEM((CS,), jnp.int32),
            pltpu.SemaphoreType.DMA,
        ],
    )
    return f(msg_pp, msg_lp, msg_pl, d_pp, d_lp, d_pl)


def _gate_body(z_ref, ea_ref, we_ref, o_ref):
    z = z_ref[...] + jnp.dot(ea_ref[...], we_ref[...],
                             preferred_element_type=jnp.float32)
    zf = z[:, :H]
    zs = z[:, H:]
    o_ref[...] = jax.nn.sigmoid(zf) * jax.nn.softplus(zs)


def _gate(zcat, ea, wecat, blk):
    e = zcat.shape[0]
    grid = e // blk
    return pl.pallas_call(
        _gate_body,
        grid=(grid,),
        in_specs=[pl.BlockSpec((blk, ZC), lambda i: (i, 0)),
                  pl.BlockSpec((blk, NG), lambda i: (i, 0)),
                  pl.BlockSpec((NG, ZC), lambda i: (0, 0))],
        out_specs=pl.BlockSpec((blk, H), lambda i: (i, 0)),
        out_shape=jax.ShapeDtypeStruct((e, H), jnp.float32),
    )(zcat, ea, wecat)


def _smear(dist):
    offset = jnp.linspace(0.0, 8.0, NG)
    coeff = -0.5 / ((8.0 - 0.0) / (NG - 1)) ** 2
    d = dist.reshape(-1, 1) - offset.reshape(1, -1)
    return jnp.exp(coeff * d * d)


def _ln(x, g, b):
    mu = jnp.mean(x, axis=-1, keepdims=True)
    var = jnp.var(x, axis=-1, keepdims=True)
    return (x - mu) / jnp.sqrt(var + 1e-5) * g + b


def _post(agg, x_dst, p):
    agg = agg / jnp.sqrt(1.0 + 1e-5) * p['bn_g'] + p['bn_b']
    out = jax.nn.relu(_ln(agg + x_dst, p['ln_g'], p['ln_b']))
    return out + x_dst


def _proj(x, p, part):
    lo = 0 if part == 'i' else H
    w = jnp.concatenate([p['Wf'][lo:lo + H], p['Ws'][lo:lo + H]], axis=1)
    t = x @ w
    if part == 'i':
        t = t + jnp.concatenate([p['bf'], p['bs']])
    return t


def _pad_edges(ei, n_pad, dst_pad_base):
    e = ei.shape[1]
    if e == n_pad:
        return ei[0], ei[1]
    pad = n_pad - e
    src = jnp.concatenate([ei[0], jnp.zeros((pad,), jnp.int32)])
    dst = jnp.concatenate(
        [ei[1], dst_pad_base + (jnp.arange(pad, dtype=jnp.int32) % 8)])
    return src, dst


def kernel(x_protein, x_ligand, edge_index_pp, edge_index_lp, edge_index_pl,
           edge_attr_pp, edge_attr_lp, edge_attr_pl, params):
    hp = _pad_rows(x_protein, NP_PAD) @ params['Wp'] + params['bp']
    hl = _pad_rows(x_ligand, NL_PAD) @ params['Wl'] + params['bl']

    src_pp, dst_pp = _pad_edges(edge_index_pp, E_PP_PAD, 10000)
    src_lp, dst_lp = _pad_edges(edge_index_lp, E_LP_PAD, 10000)
    src_pl, dst_pl = _pad_edges(edge_index_pl, E_PL_PAD, 2000)

    ea_pp = _smear(jnp.pad(edge_attr_pp, (0, E_PP_PAD - edge_attr_pp.shape[0])))
    ea_lp = _smear(jnp.pad(edge_attr_lp, (0, E_LP_PAD - edge_attr_lp.shape[0])))
    ea_pl = _smear(jnp.pad(edge_attr_pl, (0, E_PL_PAD - edge_attr_pl.shape[0])))

    dst_pl_off = dst_pl + NP_PAD

    idxs2d = [x.reshape(NWORK, -1, CE) for x in
              (dst_pp, src_pp, dst_lp, src_lp, dst_pl, src_pl)]
    e_pads = (E_PP_PAD, E_LP_PAD, E_PL_PAD)

    for lp in params['layers']:
        z_pp = _sc_gather1(_proj(hp, lp['pp'], 'i'), _proj(hp, lp['pp'], 'j'),
                           idxs2d[0], idxs2d[1], E_PP_PAD)
        z_lp = _sc_gather1(_proj(hp, lp['lp'], 'i'), _proj(hl, lp['lp'], 'j'),
                           idxs2d[2], idxs2d[3], E_LP_PAD)
        z_pl = _sc_gather1(_proj(hl, lp['pl'], 'i'), _proj(hp, lp['pl'], 'j'),
                           idxs2d[4], idxs2d[5], E_PL_PAD)

        we_pp = jnp.concatenate([lp['pp']['Wf'][2 * H:], lp['pp']['Ws'][2 * H:]], 1)
        we_lp = jnp.concatenate([lp['lp']['Wf'][2 * H:], lp['lp']['Ws'][2 * H:]], 1)
        we_pl = jnp.concatenate([lp['pl']['Wf'][2 * H:], lp['pl']['Ws'][2 * H:]], 1)

        msg_pp = _gate(z_pp, ea_pp, we_pp, 640)
        msg_lp = _gate(z_lp, ea_lp, we_lp, 640)
        msg_pl = _gate(z_pl, ea_pl, we_pl, 640)

        agg_pp, agg_lp, agg_pl = _sc_scatter(
            msg_pp, msg_lp, msg_pl, dst_pp, dst_lp, dst_pl_off)

        new_p = _post(agg_pp, hp, lp['pp']) + _post(agg_lp, hp, lp['lp'])
        new_l = _post(agg_pl, hl, lp['pl'])
        hp, hl = new_p, new_l

    hp = _ln(hp[:10000], params['lno_g'], params['lno_b'])
    return hp @ params['fcW'] + params['fcb']


# speedup vs baseline: 1.6391x; 1.0365x over previous
import functools

import jax
import jax.numpy as jnp
from jax import lax
from jax.experimental import pallas as pl
from jax.experimental.pallas import tpu as pltpu
from jax.experimental.pallas import tpu_sc as plsc

H = 128
NG = 16
ZC = 2 * H

SC_NC = 2
SC_NS = 16
NWORK = SC_NC * SC_NS

CE = 40
NBUF = 3

NP_PAD = 10240
NL_PAD = 2048
E_PP_PAD = 161280
E_LP_PAD = 53760
E_PL_PAD = 53760


def _pad_rows(x, n):
    return jnp.pad(x, ((0, n - x.shape[0]), (0, 0)))


def _sc_gather_body(nch, pd, ps, dj, sj, zout,
                    idxd, idxs, a_v, b_v, o_v, sema, semb, semo):
    cid = lax.axis_index("c")
    sid = lax.axis_index("s")
    wid = sid * SC_NC + cid

    ngroups = nch // NBUF
    pltpu.sync_copy(dj.at[wid], idxd)
    pltpu.sync_copy(sj.at[wid], idxs)

    for sl in range(NBUF):
        pltpu.async_copy(pd.at[idxd.at[sl]], a_v[sl], sema.at[sl])
        pltpu.async_copy(ps.at[idxs.at[sl]], b_v[sl], semb.at[sl])

    def group_body(g, _):
        for sl in range(NBUF):
            c = g * NBUF + sl
            pltpu.make_async_copy(pd.at[idxd.at[sl]], a_v[sl],
                                  sema.at[sl]).wait()
            pltpu.make_async_copy(ps.at[idxs.at[sl]], b_v[sl],
                                  semb.at[sl]).wait()

            @pl.when(g > 0)
            def _():
                pltpu.make_async_copy(o_v[sl], zout.at[pl.ds(0, CE)],
                                      semo.at[sl]).wait()

            def add_body(e, _):
                for v in range(ZC // 16):
                    vsl = pl.ds(v * 16, 16)
                    o_v[sl][e, vsl] = a_v[sl][e, vsl] + b_v[sl][e, vsl]
                return 0

            lax.fori_loop(0, CE, add_body, 0)

            @pl.when(c + NBUF < nch)
            def _():
                pltpu.async_copy(pd.at[idxd.at[c + NBUF]], a_v[sl],
                                 sema.at[sl])
                pltpu.async_copy(ps.at[idxs.at[c + NBUF]], b_v[sl],
                                 semb.at[sl])

            row0 = (wid * nch + c) * CE
            pltpu.async_copy(o_v[sl], zout.at[pl.ds(row0, CE)],
                             semo.at[sl])
        return 0

    lax.fori_loop(0, ngroups, group_body, 0)

    for sl in range(NBUF):
        pltpu.make_async_copy(o_v[sl], zout.at[pl.ds(0, CE)],
                              semo.at[sl]).wait()


def _sc_gather1(pd, ps, dj, sj, e_pad):
    nch = e_pad // (NWORK * CE)
    f = pl.kernel(
        functools.partial(_sc_gather_body, nch),
        out_type=jax.ShapeDtypeStruct((e_pad, ZC), jnp.float32),
        mesh=plsc.VectorSubcoreMesh(core_axis_name="c", subcore_axis_name="s",
                                    num_cores=SC_NC, num_subcores=SC_NS),
        scratch_types=[
            pltpu.VMEM((nch, CE), jnp.int32),
            pltpu.VMEM((nch, CE), jnp.int32),
            [pltpu.VMEM((CE, ZC), jnp.float32)] * NBUF,
            [pltpu.VMEM((CE, ZC), jnp.float32)] * NBUF,
            [pltpu.VMEM((CE, ZC), jnp.float32)] * NBUF,
            pltpu.SemaphoreType.DMA((NBUF,)),
            pltpu.SemaphoreType.DMA((NBUF,)),
            pltpu.SemaphoreType.DMA((NBUF,)),
        ],
    )
    return f(pd, ps, dj, sj)


CS = 160
AGG_ROWS = NP_PAD + NL_PAD


def _sc_scatter_body(msg_pp, msg_lp, msg_pl, d_pp, d_lp, d_pl,
                     agg_pp, agg_lp, agg_pl,
                     spmem, zbuf, mbuf, ibuf, sem):
    cid = lax.axis_index("c")
    tid = lax.axis_index("s")

    def zrow(e, _):
        for v in range(H // 16):
            zbuf[e, pl.ds(v * 16, 16)] = jnp.zeros((16,), jnp.float32)
        return 0
    lax.fori_loop(0, 64, zrow, 0)
    nz = AGG_ROWS // (16 * 64)
    for k in range(nz):
        pltpu.sync_copy(zbuf, spmem.at[pl.ds((tid * nz + k) * 64, 64)])
    plsc.subcore_barrier()

    def scatter_edges(msg, dj, nch):
        def chunk_body(c, _):
            base = (tid * nch + c) * CS
            cpi = pltpu.async_copy(dj.at[pl.ds(base, CS)], ibuf, sem)
            cpm = pltpu.async_copy(msg.at[pl.ds(base, CS)], mbuf, sem)
            cpi.wait()
            cpm.wait()
            pltpu.sync_copy(mbuf, spmem.at[ibuf], add=True)
            return 0
        lax.fori_loop(0, nch, chunk_body, 0)

    @pl.when(cid == 0)
    def _():
        scatter_edges(msg_pp, d_pp, E_PP_PAD // (SC_NS * CS))

    @pl.when(cid == 1)
    def _():
        scatter_edges(msg_lp, d_lp, E_LP_PAD // (SC_NS * CS))
        scatter_edges(msg_pl, d_pl, E_PL_PAD // (SC_NS * CS))

    plsc.subcore_barrier()

    def writeout(out_hbm, spm_base, rows_per_tile):
        for k in range(rows_per_tile // 64):
            r0 = spm_base + tid * rows_per_tile + k * 64
            o0 = tid * rows_per_tile + k * 64
            pltpu.sync_copy(spmem.at[pl.ds(r0, 64)], zbuf)
            pltpu.sync_copy(zbuf, out_hbm.at[pl.ds(o0, 64)])

    @pl.when(cid == 0)
    def _():
        writeout(agg_pp, 0, NP_PAD // SC_NS)

    @pl.when(cid == 1)
    def _():
        writeout(agg_lp, 0, NP_PAD // SC_NS)
        writeout(agg_pl, NP_PAD, NL_PAD // SC_NS)


def _sc_scatter(msg_pp, msg_lp, msg_pl, d_pp, d_lp, d_pl):
    out_type = [jax.ShapeDtypeStruct((NP_PAD, H), jnp.float32),
                jax.ShapeDtypeStruct((NP_PAD, H), jnp.float32),
                jax.ShapeDtypeStruct((NL_PAD, H), jnp.float32)]
    f = pl.kernel(
        _sc_scatter_body,
        out_type=out_type,
        mesh=plsc.VectorSubcoreMesh(core_axis_name="c", subcore_axis_name="s",
                                    num_cores=SC_NC, num_subcores=SC_NS),
        scratch_types=[
            pltpu.VMEM_SHARED((AGG_ROWS, H), jnp.float32),
            pltpu.VMEM((64, H), jnp.float32),
            pltpu.VMEM((CS, H), jnp.float32),
            pltpu.VMEM((CS,), jnp.int32),
            pltpu.SemaphoreType.DMA,
        ],
    )
    return f(msg_pp, msg_lp, msg_pl, d_pp, d_lp, d_pl)


def _gate_body(z_ref, ea_ref, we_ref, o_ref):
    z = z_ref[...] + jnp.dot(ea_ref[...], we_ref[...],
                             preferred_element_type=jnp.float32)
    zf = z[:, :H]
    zs = z[:, H:]
    o_ref[...] = jax.nn.sigmoid(zf) * jax.nn.softplus(zs)


def _gate(zcat, ea, wecat, blk):
    e = zcat.shape[0]
    grid = e // blk
    return pl.pallas_call(
        _gate_body,
        grid=(grid,),
        in_specs=[pl.BlockSpec((blk, ZC), lambda i: (i, 0)),
                  pl.BlockSpec((blk, NG), lambda i: (i, 0)),
                  pl.BlockSpec((NG, ZC), lambda i: (0, 0))],
        out_specs=pl.BlockSpec((blk, H), lambda i: (i, 0)),
        out_shape=jax.ShapeDtypeStruct((e, H), jnp.float32),
    )(zcat, ea, wecat)


def _smear(dist):
    offset = jnp.linspace(0.0, 8.0, NG)
    coeff = -0.5 / ((8.0 - 0.0) / (NG - 1)) ** 2
    d = dist.reshape(-1, 1) - offset.reshape(1, -1)
    return jnp.exp(coeff * d * d)


def _ln(x, g, b):
    mu = jnp.mean(x, axis=-1, keepdims=True)
    var = jnp.var(x, axis=-1, keepdims=True)
    return (x - mu) / jnp.sqrt(var + 1e-5) * g + b


def _post(agg, x_dst, p):
    agg = agg / jnp.sqrt(1.0 + 1e-5) * p['bn_g'] + p['bn_b']
    out = jax.nn.relu(_ln(agg + x_dst, p['ln_g'], p['ln_b']))
    return out + x_dst


def _proj(x, p, part):
    lo = 0 if part == 'i' else H
    w = jnp.concatenate([p['Wf'][lo:lo + H], p['Ws'][lo:lo + H]], axis=1)
    t = x @ w
    if part == 'i':
        t = t + jnp.concatenate([p['bf'], p['bs']])
    return t


def _pad_edges(ei, n_pad, dst_pad_base):
    e = ei.shape[1]
    if e == n_pad:
        return ei[0], ei[1]
    pad = n_pad - e
    src = jnp.concatenate([ei[0], jnp.zeros((pad,), jnp.int32)])
    dst = jnp.concatenate(
        [ei[1], dst_pad_base + (jnp.arange(pad, dtype=jnp.int32) % 8)])
    return src, dst


def kernel(x_protein, x_ligand, edge_index_pp, edge_index_lp, edge_index_pl,
           edge_attr_pp, edge_attr_lp, edge_attr_pl, params):
    hp = _pad_rows(x_protein, NP_PAD) @ params['Wp'] + params['bp']
    hl = _pad_rows(x_ligand, NL_PAD) @ params['Wl'] + params['bl']

    src_pp, dst_pp = _pad_edges(edge_index_pp, E_PP_PAD, 10000)
    src_lp, dst_lp = _pad_edges(edge_index_lp, E_LP_PAD, 10000)
    src_pl, dst_pl = _pad_edges(edge_index_pl, E_PL_PAD, 2000)

    ea_pp = _smear(jnp.pad(edge_attr_pp, (0, E_PP_PAD - edge_attr_pp.shape[0])))
    ea_lp = _smear(jnp.pad(edge_attr_lp, (0, E_LP_PAD - edge_attr_lp.shape[0])))
    ea_pl = _smear(jnp.pad(edge_attr_pl, (0, E_PL_PAD - edge_attr_pl.shape[0])))

    dst_pl_off = dst_pl + NP_PAD

    idxs2d = [x.reshape(NWORK, -1, CE) for x in
              (dst_pp, src_pp, dst_lp, src_lp, dst_pl, src_pl)]
    e_pads = (E_PP_PAD, E_LP_PAD, E_PL_PAD)

    for lp in params['layers']:
        z_pp = _sc_gather1(_proj(hp, lp['pp'], 'i'), _proj(hp, lp['pp'], 'j'),
                           idxs2d[0], idxs2d[1], E_PP_PAD)
        z_lp = _sc_gather1(_proj(hp, lp['lp'], 'i'), _proj(hl, lp['lp'], 'j'),
                           idxs2d[2], idxs2d[3], E_LP_PAD)
        z_pl = _sc_gather1(_proj(hl, lp['pl'], 'i'), _proj(hp, lp['pl'], 'j'),
                           idxs2d[4], idxs2d[5], E_PL_PAD)

        we_pp = jnp.concatenate([lp['pp']['Wf'][2 * H:], lp['pp']['Ws'][2 * H:]], 1)
        we_lp = jnp.concatenate([lp['lp']['Wf'][2 * H:], lp['lp']['Ws'][2 * H:]], 1)
        we_pl = jnp.concatenate([lp['pl']['Wf'][2 * H:], lp['pl']['Ws'][2 * H:]], 1)

        msg_pp = _gate(z_pp, ea_pp, we_pp, 640)
        msg_lp = _gate(z_lp, ea_lp, we_lp, 640)
        msg_pl = _gate(z_pl, ea_pl, we_pl, 640)

        agg_pp, agg_lp, agg_pl = _sc_scatter(
            msg_pp, msg_lp, msg_pl, dst_pp, dst_lp, dst_pl_off)

        new_p = _post(agg_pp, hp, lp['pp']) + _post(agg_lp, hp, lp['lp'])
        new_l = _post(agg_pl, hl, lp['pl'])
        hp, hl = new_p, new_l

    hp = _ln(hp[:10000], params['lno_g'], params['lno_b'])
    return hp @ params['fcW'] + params['fcb']
